# baseline (speedup 1.0000x reference)
import functools

import jax
import jax.numpy as jnp
from jax import lax
from jax.experimental import pallas as pl
from jax.experimental.pallas import tpu as pltpu
from jax.experimental.pallas import tpu_sc as plsc

VOCAB = 1000000
EMBED_DIM = 32
BATCH = 4096
HIST = 200
TOTAL = BATCH * HIST

NUM_CORES = 2
NUM_SUBCORES = 16
NW = NUM_CORES * NUM_SUBCORES
BBLK = BATCH // NW
HWIN = 4
NWIN = HIST // HWIN
PWIN = HWIN * EMBED_DIM
ROWS = HWIN * BBLK

_mesh = plsc.VectorSubcoreMesh(
    core_axis_name="c", subcore_axis_name="s",
    num_cores=NUM_CORES, num_subcores=NUM_SUBCORES)


@functools.partial(
    pl.kernel,
    out_type=jax.ShapeDtypeStruct((TOTAL // (8 * 128), NW, 8, 128),
                                  jnp.float32),
    mesh=_mesh,
    scratch_types=[
        pltpu.VMEM((2, HWIN, BBLK), jnp.int32),
        pltpu.VMEM((2, ROWS, EMBED_DIM), jnp.float32),
        pltpu.VMEM((2, PWIN // 8, 8, 128), jnp.float32),
        pltpu.SemaphoreType.DMA,
        pltpu.SemaphoreType.DMA,
    ],
    compiler_params=pltpu.CompilerParams(
        use_tc_tiling_on_sc=False, needs_layout_passes=False),
)
def _emb_lookup(idxT_hbm, table_hbm, out_hbm, idx_v, rows_v, outv, gsem, osem):
    wid = lax.axis_index("s") * NUM_CORES + lax.axis_index("c")
    b0 = wid * BBLK

    def stage(win, buf):
        pltpu.sync_copy(
            idxT_hbm.at[pl.ds(win * HWIN, HWIN), pl.ds(b0, BBLK)],
            idx_v.at[buf])
        for j in range(HWIN):
            pltpu.async_copy(
                table_hbm.at[idx_v.at[buf, j]],
                rows_v.at[buf, pl.ds(j * BBLK, BBLK)],
                gsem,
            )

    def drain_gathers(buf):
        pltpu.make_async_copy(
            table_hbm.at[pl.ds(0, ROWS)], rows_v.at[buf], gsem).wait()

    def transpose(buf):
        iota16 = lax.iota(jnp.int32, 16)

        @pl.loop(0, PWIN)
        def _row(p):
            row_base = (p // EMBED_DIM) * BBLK
            col = jnp.full((16,), p % EMBED_DIM, jnp.int32)
            for v in range(BBLK // 16):
                rows_idx = iota16 + (row_base + v * 16)
                vals = plsc.load_gather(rows_v.at[buf], [rows_idx, col])
                outv[buf, p // 8, p % 8, pl.ds(v * 16, 16)] = vals

    def writeback(win, buf):
        pltpu.async_copy(
            outv.at[buf],
            out_hbm.at[pl.ds(win * (PWIN // 8), PWIN // 8), wid],
            osem)

    def drain_out(buf):
        pltpu.make_async_copy(
            outv.at[buf], out_hbm.at[pl.ds(0, PWIN // 8), 0], osem).wait()

    stage(0, 0)

    @pl.loop(0, NWIN // 2)
    def _pair(pr):
        w0 = 2 * pr
        for t in range(2):
            win = w0 + t
            buf = t
            nbuf = 1 - t

            @pl.when(win + 1 < NWIN)
            def _prefetch():
                stage(win + 1, nbuf)

            drain_gathers(buf)

            @pl.when(win >= 2)
            def _free():
                drain_out(buf)

            transpose(buf)
            writeback(win, buf)

    drain_out(0)
    drain_out(1)


@jax.jit
def kernel(w_tensor, table):
    idxT = w_tensor.astype(jnp.int32).T
    out4d = _emb_lookup(idxT, table)
    out = out4d.transpose(1, 3, 0, 2).reshape(BATCH, HIST * EMBED_DIM)
    return out.reshape(BATCH, HIST, EMBED_DIM)


# speedup vs baseline: 1.0002x; 1.0002x over previous
import functools

import jax
import jax.numpy as jnp
from jax import lax
from jax.experimental import pallas as pl
from jax.experimental.pallas import tpu as pltpu
from jax.experimental.pallas import tpu_sc as plsc

VOCAB = 1000000
EMBED_DIM = 32
BATCH = 4096
HIST = 200
TOTAL = BATCH * HIST

NUM_CORES = 2
NUM_SUBCORES = 16
NW = NUM_CORES * NUM_SUBCORES
BBLK = BATCH // NW
HWIN = 4
NWIN = HIST // HWIN
PWIN = HWIN * EMBED_DIM
ROWS = HWIN * BBLK

_mesh = plsc.VectorSubcoreMesh(
    core_axis_name="c", subcore_axis_name="s",
    num_cores=NUM_CORES, num_subcores=NUM_SUBCORES)


@functools.partial(
    pl.kernel,
    out_type=jax.ShapeDtypeStruct((TOTAL // (8 * 128), NW, 8, 128),
                                  jnp.float32),
    mesh=_mesh,
    scratch_types=[
        pltpu.VMEM((2, HWIN, BBLK), jnp.int32),
        pltpu.VMEM((2, ROWS, EMBED_DIM), jnp.float32),
        pltpu.VMEM((2, PWIN // 8, 8, 128), jnp.float32),
        pltpu.SemaphoreType.DMA,
        pltpu.SemaphoreType.DMA,
    ],
    compiler_params=pltpu.CompilerParams(
        use_tc_tiling_on_sc=False, needs_layout_passes=False),
)
def _emb_lookup(idxT_hbm, table_hbm, out_hbm, idx_v, rows_v, outv, gsem, osem):
    wid = lax.axis_index("s") * NUM_CORES + lax.axis_index("c")
    b0 = wid * BBLK

    def stage(win, buf):
        pltpu.sync_copy(
            idxT_hbm.at[pl.ds(win * HWIN, HWIN), pl.ds(b0, BBLK)],
            idx_v.at[buf])
        for j in range(HWIN):
            pltpu.async_copy(
                table_hbm.at[idx_v.at[buf, j]],
                rows_v.at[buf, pl.ds(j * BBLK, BBLK)],
                gsem,
            )

    def drain_gathers(buf):
        pltpu.make_async_copy(
            table_hbm.at[pl.ds(0, ROWS)], rows_v.at[buf], gsem).wait()

    iota16 = lax.iota(jnp.int32, 16)
    row_vecs = [[iota16 + (h * BBLK + 16 * v) for v in range(BBLK // 16)]
                for h in range(HWIN)]

    def transpose(buf):
        @pl.loop(0, EMBED_DIM)
        def _col(c):
            col = jnp.full((16,), c, jnp.int32)
            a = c // 8
            s = c % 8
            for h in range(HWIN):
                for v in range(BBLK // 16):
                    vals = plsc.load_gather(
                        rows_v.at[buf], [row_vecs[h][v], col])
                    outv[buf, h * (EMBED_DIM // 8) + a, s,
                         pl.ds(v * 16, 16)] = vals

    def writeback(win, buf):
        pltpu.async_copy(
            outv.at[buf],
            out_hbm.at[pl.ds(win * (PWIN // 8), PWIN // 8), wid],
            osem)

    def drain_out(buf):
        pltpu.make_async_copy(
            outv.at[buf], out_hbm.at[pl.ds(0, PWIN // 8), 0], osem).wait()

    stage(0, 0)

    @pl.loop(0, NWIN // 2)
    def _pair(pr):
        w0 = 2 * pr
        for t in range(2):
            win = w0 + t
            buf = t
            nbuf = 1 - t

            @pl.when(win + 1 < NWIN)
            def _prefetch():
                stage(win + 1, nbuf)

            drain_gathers(buf)

            @pl.when(win >= 2)
            def _free():
                drain_out(buf)

            transpose(buf)
            writeback(win, buf)

    drain_out(0)
    drain_out(1)


@jax.jit
def kernel(w_tensor, table):
    idxT = w_tensor.astype(jnp.int32).T
    out4d = _emb_lookup(idxT, table)
    out = out4d.transpose(1, 3, 0, 2).reshape(BATCH, HIST * EMBED_DIM)
    return out.reshape(BATCH, HIST, EMBED_DIM)


# speedup vs baseline: 1.1166x; 1.1164x over previous
import functools

import jax
import jax.numpy as jnp
from jax import lax
from jax.experimental import pallas as pl
from jax.experimental.pallas import tpu as pltpu
from jax.experimental.pallas import tpu_sc as plsc

VOCAB = 1000000
EMBED_DIM = 32
BATCH = 4096
HIST = 200
TOTAL = BATCH * HIST

NUM_CORES = 2
NUM_SUBCORES = 16
NW = NUM_CORES * NUM_SUBCORES
BBLK = BATCH // NW
HWIN = 4
NWIN = HIST // HWIN
PWIN = HWIN * EMBED_DIM
ROWS = HWIN * BBLK

_mesh = plsc.VectorSubcoreMesh(
    core_axis_name="c", subcore_axis_name="s",
    num_cores=NUM_CORES, num_subcores=NUM_SUBCORES)


@functools.partial(
    pl.kernel,
    out_type=jax.ShapeDtypeStruct((TOTAL // (8 * 128), NW, 8, 128),
                                  jnp.float32),
    mesh=_mesh,
    scratch_types=[
        pltpu.VMEM((2, HWIN, BBLK), jnp.int32),
        pltpu.VMEM((2, ROWS, EMBED_DIM), jnp.float32),
        pltpu.VMEM((2, PWIN // 8, 8, 128), jnp.float32),
        pltpu.SemaphoreType.DMA,
        pltpu.SemaphoreType.DMA,
    ],
    compiler_params=pltpu.CompilerParams(
        use_tc_tiling_on_sc=False, needs_layout_passes=False),
)
def _emb_lookup(idxT_hbm, table_hbm, out_hbm, idx_v, rows_v, outv, gsem, osem):
    wid = lax.axis_index("s") * NUM_CORES + lax.axis_index("c")
    b0 = wid * BBLK

    def stage(win, buf):
        pltpu.sync_copy(
            idxT_hbm.at[pl.ds(win * HWIN, HWIN), pl.ds(b0, BBLK)],
            idx_v.at[buf])
        for j in range(HWIN):
            pltpu.async_copy(
                table_hbm.at[idx_v.at[buf, j]],
                rows_v.at[buf, pl.ds(j * BBLK, BBLK)],
                gsem,
            )

    def drain_gathers(buf):
        pltpu.make_async_copy(
            table_hbm.at[pl.ds(0, ROWS)], rows_v.at[buf], gsem).wait()

    iota16 = lax.iota(jnp.int32, 16)
    row_vecs = [[iota16 + (h * BBLK + 16 * v) for v in range(BBLK // 16)]
                for h in range(HWIN)]

    def transpose(buf):
        @pl.loop(0, EMBED_DIM, unroll=2)
        def _col(c):
            col = jnp.full((16,), c, jnp.int32)
            a = c // 8
            s = c % 8
            for h in range(HWIN):
                vals = [
                    plsc.load_gather(rows_v.at[buf], [row_vecs[h][v], col])
                    for v in range(BBLK // 16)
                ]
                for v in range(BBLK // 16):
                    outv[buf, h * (EMBED_DIM // 8) + a, s,
                         pl.ds(v * 16, 16)] = vals[v]

    def writeback(win, buf):
        pltpu.async_copy(
            outv.at[buf],
            out_hbm.at[pl.ds(win * (PWIN // 8), PWIN // 8), wid],
            osem)

    def drain_out(buf):
        pltpu.make_async_copy(
            outv.at[buf], out_hbm.at[pl.ds(0, PWIN // 8), 0], osem).wait()

    stage(0, 0)

    @pl.loop(0, NWIN // 2)
    def _pair(pr):
        w0 = 2 * pr
        for t in range(2):
            win = w0 + t
            buf = t
            nbuf = 1 - t

            @pl.when(win + 1 < NWIN)
            def _prefetch():
                stage(win + 1, nbuf)

            drain_gathers(buf)

            @pl.when(win >= 2)
            def _free():
                drain_out(buf)

            transpose(buf)
            writeback(win, buf)

    drain_out(0)
    drain_out(1)


@jax.jit
def kernel(w_tensor, table):
    idxT = w_tensor.astype(jnp.int32).T
    out4d = _emb_lookup(idxT, table)
    out = out4d.transpose(1, 3, 0, 2).reshape(BATCH, HIST * EMBED_DIM)
    return out.reshape(BATCH, HIST, EMBED_DIM)
